# Initial kernel scaffold; baseline (speedup 1.0000x reference)
import jax
import jax.numpy as jnp
from jax.experimental import pallas as pl


def kernel(actions, emb_table, base_emb):
    raise NotImplementedError("write your pallas kernel here")


# speedup vs baseline: 3.4710x; 3.4710x over previous
import functools

import jax
import jax.numpy as jnp
from jax import lax
from jax.experimental import pallas as pl
from jax.experimental.pallas import tpu as pltpu
from jax.experimental.pallas import tpu_sc as plsc

D_MODEL = 64
B = 4096
T = 200

NC = 2
NS = 16
NW = NC * NS

TOTAL = B * T
PER_W = TOTAL // NW
GATHER = 128
CHUNK = 1024
N_GATHER = CHUNK // GATHER
N_CHUNK = PER_W // CHUNK
IDX_ROWS = PER_W // GATHER


def _bias_body(table_ref, base_ref, out_ref):
    out_ref[...] = table_ref[...] + base_ref[...]


def _bias_table(emb_table, base_emb):
    return pl.pallas_call(
        _bias_body,
        out_shape=jax.ShapeDtypeStruct(emb_table.shape, emb_table.dtype),
    )(emb_table, base_emb.reshape(1, D_MODEL))


def _gather_body(table_hbm, idx_hbm, out_hbm, idx_v, rows_v, sem):
    wid = lax.axis_index("s") * NC + lax.axis_index("c")
    pltpu.sync_copy(idx_hbm.at[wid], idx_v)

    def chunk_body(j, _):
        copies = []
        for b in range(N_GATHER):
            copies.append(
                pltpu.async_copy(
                    table_hbm.at[idx_v.at[j * N_GATHER + b]],
                    rows_v.at[pl.ds(b * GATHER, GATHER)],
                    sem,
                )
            )
        for c in copies:
            c.wait()
        pltpu.sync_copy(
            rows_v,
            out_hbm.at[pl.ds(wid * PER_W + j * CHUNK, CHUNK)],
        )
        return ()

    lax.fori_loop(0, N_CHUNK, chunk_body, (), unroll=False)


@jax.jit
def kernel(actions, emb_table, base_emb):
    biased = _bias_table(emb_table, base_emb)
    idx = actions.astype(jnp.int32).reshape(NW, IDX_ROWS, GATHER)

    mesh = plsc.VectorSubcoreMesh(core_axis_name="c", subcore_axis_name="s")
    out = pl.kernel(
        _gather_body,
        out_type=jax.ShapeDtypeStruct((TOTAL, D_MODEL), jnp.float32),
        mesh=mesh,
        scratch_types=[
            pltpu.VMEM((IDX_ROWS, GATHER), jnp.int32),
            pltpu.VMEM((CHUNK, D_MODEL), jnp.float32),
            pltpu.SemaphoreType.DMA,
        ],
        compiler_params=pltpu.CompilerParams(use_tc_tiling_on_sc=False),
    )(biased, idx)
    return out.reshape(B, T, 1, D_MODEL)
